# baseline (speedup 1.0000x reference)
import functools

import jax
import jax.numpy as jnp
from jax import lax
from jax.experimental import pallas as pl
from jax.experimental.pallas import tpu as pltpu
from jax.experimental.pallas import tpu_sc as plsc

_D = 64
_B = 16384
_V = 1000000
_TPW = 244
_MAIN = _TPW * 32 * 128
_TAIL0 = _MAIN
_SLABC = 256
_NSLAB = _TPW * 128 // _SLABC
_TRASH = _B
_BP = _B + 16

_info = plsc.get_sparse_core_info()
_NC = _info.num_cores
_NS = _info.num_subcores
_NW = _NC * _NS

_mesh = plsc.VectorSubcoreMesh(core_axis_name="c", subcore_axis_name="s")


@functools.partial(
    pl.kernel,
    mesh=_mesh,
    out_type=jax.ShapeDtypeStruct((_BP, 128), jnp.float32),
    scratch_types=[
        pltpu.VMEM((_B + 16,), jnp.int32),
        pltpu.VMEM((_B + 16,), jnp.int32),
        pltpu.VMEM_SHARED((_NS, 2, _D, _SLABC), jnp.float32),
        pltpu.VMEM((2, _D, _SLABC), jnp.float32),
        pltpu.VMEM((2, 16, 128), jnp.float32),
        pltpu.SemaphoreType.DMA,
        pltpu.SemaphoreType.DMA,
    ],
    compiler_params=pltpu.CompilerParams(needs_layout_passes=False),
)
def _gather_kernel(table_hbm, tail_hbm, idx_hbm, out_hbm, idxb, mp,
                   slab_sh, slab_v, stage_v, sem, osem):
    sid = lax.axis_index("s")
    wid = sid * _NC + lax.axis_index("c")
    lo = wid * (_TPW * 128)
    is_last = wid == (_NW - 1)
    hi = jnp.where(is_last, _V, lo + _TPW * 128)
    sp = idxb

    pltpu.sync_copy(idx_hbm, idxb.at[pl.ds(0, _B)])

    lanes = lax.iota(jnp.int32, 16)
    zeros16 = jnp.full((16,), 0, jnp.int32)

    def scan_body(i, cnt):
        v = idxb[pl.ds(i * 16, 16)]
        m = jnp.logical_and(v >= lo, v < hi)
        packed = lax.shift_left(v - lo, 14) + (lanes + i * 16)
        plsc.store_compressed(mp.at[pl.ds(cnt, 16)], packed, mask=m)
        return cnt + jnp.max(plsc.all_reduce_population_count(m))

    cnt = lax.fori_loop(0, _B // 16, scan_body, jnp.int32(0))
    nwin = lax.shift_right_logical(cnt + 15, 4)

    def process_slab(rel_lo, slab_w, src_ref):
        def filt_body(i, c2):
            p = mp[pl.ds(i * 16, 16)]
            r = lax.shift_right_logical(p, 14)
            valid = (lanes + i * 16) < cnt
            m = jnp.logical_and(
                jnp.logical_and(r >= rel_lo, r < rel_lo + slab_w), valid
            )
            plsc.store_compressed(sp.at[pl.ds(c2, 16)], p, mask=m)
            return c2 + jnp.max(plsc.all_reduce_population_count(m))

        c2 = lax.fori_loop(0, nwin, filt_body, jnp.int32(0))
        nwin2 = lax.shift_right_logical(c2 + 15, 4)

        def win_body(w, carry):
            p = sp[pl.ds(w * 16, 16)]
            r = lax.shift_right_logical(p, 14)
            valid = (lanes + w * 16) < c2
            bsafe = jnp.where(
                valid, lax.bitwise_and(p, (1 << 14) - 1), _TRASH
            )
            m = jnp.minimum(
                jnp.maximum(r - rel_lo, zeros16), slab_w - 1
            )
            buf = lax.rem(w, 2)

            @pl.when(w >= 2)
            def _():
                pltpu.make_async_copy(
                    out_hbm.at[pl.ds(0, 16)], stage_v.at[buf], osem
                ).wait()

            for c in range(_D):
                cvec = zeros16 + c
                vals = plsc.load_gather(src_ref, [cvec, m])
                plsc.store_scatter(stage_v.at[buf], [lanes, cvec], vals)
            pltpu.async_copy(stage_v.at[buf], out_hbm.at[bsafe], osem)
            return carry

        lax.fori_loop(0, nwin2, win_body, jnp.int32(0))

        def drain_body(i, carry):
            pltpu.make_async_copy(
                out_hbm.at[pl.ds(0, 16)], stage_v.at[0], osem
            ).wait()
            return carry

        lax.fori_loop(0, jnp.minimum(nwin2, 2), drain_body, jnp.int32(0))

    def slab_start(s, buf):
        col = lo + s * _SLABC
        for g in range(_D // 8):
            pltpu.async_copy(
                table_hbm.at[pl.ds(g * 8, 8), pl.ds(col, _SLABC)],
                slab_sh.at[sid, buf, pl.ds(g * 8, 8), :],
                sem,
            )

    def slab_wait(s, buf):
        col = lo + s * _SLABC
        for g in range(_D // 8):
            pltpu.make_async_copy(
                table_hbm.at[pl.ds(g * 8, 8), pl.ds(col, _SLABC)],
                slab_sh.at[sid, buf, pl.ds(g * 8, 8), :],
                sem,
            ).wait()

    slab_start(0, 0)

    def stream_body(s, carry):
        buf = lax.rem(s, 2)

        @pl.when(s + 1 < _NSLAB)
        def _():
            slab_start(s + 1, 1 - buf)

        slab_wait(s, buf)
        pltpu.sync_copy(slab_sh.at[sid, buf], slab_v.at[buf])
        process_slab(s * _SLABC, _SLABC, slab_v.at[buf])
        return carry

    lax.fori_loop(0, _NSLAB, stream_body, jnp.int32(0))

    @pl.when(is_last)
    def _():
        for t in range(2):
            for g in range(_D // 8):
                pltpu.sync_copy(
                    table_hbm.at[
                        pl.ds(g * 8, 8), pl.ds(_TAIL0 + t * 256, 256)
                    ],
                    slab_v.at[0, pl.ds(g * 8, 8), :],
                )
            process_slab(_TAIL0 - lo + t * 256, 256, slab_v.at[0])
        pltpu.sync_copy(tail_hbm, slab_v.at[0, :, pl.ds(0, 128)])
        process_slab(_TAIL0 + 512 - lo, _D, slab_v.at[0])


def kernel(inputs, in_embed_weight):
    idx = inputs.astype(jnp.int32)
    tail = jnp.pad(
        in_embed_weight[_TAIL0 + 512:].T, ((0, 0), (0, 128 - _D))
    )
    out_p = _gather_kernel(in_embed_weight.T, tail, idx)
    return out_p[:_B, :_D]


# speedup vs baseline: 3.1013x; 3.1013x over previous
import functools

import jax
import jax.numpy as jnp
from jax import lax
from jax.experimental import pallas as pl
from jax.experimental.pallas import tpu as pltpu
from jax.experimental.pallas import tpu_sc as plsc

_D = 64
_B = 16384

_info = plsc.get_sparse_core_info()
_NC = _info.num_cores
_NS = _info.num_subcores
_NW = _NC * _NS
_BPW = _B // _NW
_CHUNK = 128
_NCHUNK = _BPW // _CHUNK

_mesh = plsc.VectorSubcoreMesh(core_axis_name="c", subcore_axis_name="s")


@functools.partial(
    pl.kernel,
    mesh=_mesh,
    out_type=jax.ShapeDtypeStruct((_B, 2 * _D), jnp.float32),
    scratch_types=[
        pltpu.VMEM((_NCHUNK, _CHUNK), jnp.int32),
        pltpu.VMEM((_BPW, 2 * _D), jnp.float32),
        pltpu.SemaphoreType.DMA,
    ],
    compiler_params=pltpu.CompilerParams(needs_layout_passes=False),
)
def _gather_kernel(pairs_hbm, idx_hbm, out_hbm, jsel_v, rows_v, sem):
    wid = lax.axis_index("s") * _NC + lax.axis_index("c")
    base = wid * _BPW
    for j in range(_NCHUNK):
        pltpu.sync_copy(
            idx_hbm.at[pl.ds(base + j * _CHUNK, _CHUNK)], jsel_v.at[j]
        )
    for j in range(_NCHUNK):
        for k in range(_CHUNK // 16):
            v = jsel_v[j, pl.ds(k * 16, 16)]
            jsel_v[j, pl.ds(k * 16, 16)] = lax.shift_right_logical(v, 1)
    copies = [
        pltpu.async_copy(
            pairs_hbm.at[jsel_v.at[j]],
            rows_v.at[pl.ds(j * _CHUNK, _CHUNK)],
            sem,
        )
        for j in range(_NCHUNK)
    ]
    for c in copies:
        c.wait()
    pltpu.sync_copy(rows_v, out_hbm.at[pl.ds(base, _BPW)])


def kernel(inputs, in_embed_weight):
    idx = inputs.astype(jnp.int32)
    pairs = in_embed_weight.reshape(500000, 2 * _D)
    out_p = _gather_kernel(pairs, idx)
    odd = (idx & 1)[:, None] == 1
    return jnp.where(odd, out_p[:, _D:], out_p[:, :_D])


# speedup vs baseline: 7.6591x; 2.4696x over previous
import functools

import jax
import jax.numpy as jnp
from jax import lax
from jax.experimental import pallas as pl
from jax.experimental.pallas import tpu as pltpu
from jax.experimental.pallas import tpu_sc as plsc

_D = 64
_B = 16384
_R = 8
_NT = 125000

_info = plsc.get_sparse_core_info()
_NC = _info.num_cores
_NS = _info.num_subcores
_NW = _NC * _NS
_BPW = _B // _NW
_K = 32
_NG = _BPW // _K

_mesh = plsc.VectorSubcoreMesh(core_axis_name="c", subcore_axis_name="s")


@functools.partial(
    pl.kernel,
    mesh=_mesh,
    out_type=jax.ShapeDtypeStruct((_B, _D), jnp.float32),
    scratch_types=[
        pltpu.SMEM((_BPW,), jnp.int32),
        pltpu.VMEM_SHARED((_NS, _BPW), jnp.int32),
        pltpu.VMEM((_BPW, _D), jnp.float32),
        pltpu.SemaphoreType.DMA,
        pltpu.SemaphoreType.DMA,
    ],
)
def _gather_kernel(table_hbm, idx_hbm, out_hbm, idx_s, idx_sh, rows_v,
                   sem_a, sem_b):
    sid = lax.axis_index("s")
    wid = sid * _NC + lax.axis_index("c")
    base = wid * _BPW
    pltpu.sync_copy(idx_hbm.at[pl.ds(base, _BPW)], idx_sh.at[sid])
    pltpu.sync_copy(idx_sh.at[sid], idx_s)

    sems = [sem_a, sem_b]
    pending = [None, None]

    for g in range(_NG):
        gb = g * _K
        sem = sems[g % 2]
        if pending[g % 2] is not None:
            for c in pending[g % 2]:
                c.wait()
        copies = []
        for k in range(_K):
            v = idx_s[gb + k]
            t = lax.shift_right_logical(v, 3)
            r = lax.bitwise_and(v, _R - 1)
            copies.append(
                pltpu.async_copy(
                    table_hbm.at[t, r], rows_v.at[gb + k], sem
                )
            )
        pending[g % 2] = copies
    for p in pending:
        if p is not None:
            for c in p:
                c.wait()
    pltpu.sync_copy(rows_v, out_hbm.at[pl.ds(base, _BPW)])


def kernel(inputs, in_embed_weight):
    idx = inputs.astype(jnp.int32)
    table3 = in_embed_weight.reshape(_NT, _R, _D)
    return _gather_kernel(table3, idx)


# speedup vs baseline: 7.7577x; 1.0129x over previous
import functools

import jax
import jax.numpy as jnp
from jax import lax
from jax.experimental import pallas as pl
from jax.experimental.pallas import tpu as pltpu
from jax.experimental.pallas import tpu_sc as plsc

_D = 64
_B = 16384
_R = 8
_NT = 125000

_info = plsc.get_sparse_core_info()
_NC = _info.num_cores
_NS = _info.num_subcores
_NW = _NC * _NS
_BPW = _B // _NW
_K = 64
_NG = _BPW // _K

_mesh = plsc.VectorSubcoreMesh(core_axis_name="c", subcore_axis_name="s")


@functools.partial(
    pl.kernel,
    mesh=_mesh,
    out_type=jax.ShapeDtypeStruct((_B, _D), jnp.float32),
    scratch_types=[
        pltpu.SMEM((_BPW,), jnp.int32),
        pltpu.VMEM_SHARED((_NS, _BPW), jnp.int32),
        pltpu.VMEM((_BPW, _D), jnp.float32),
        pltpu.SemaphoreType.DMA,
        pltpu.SemaphoreType.DMA,
    ],
)
def _gather_kernel(table_hbm, idx_hbm, out_hbm, idx_s, idx_sh, rows_v,
                   sem_a, sem_b):
    sid = lax.axis_index("s")
    wid = sid * _NC + lax.axis_index("c")
    base = wid * _BPW
    pltpu.sync_copy(idx_hbm.at[pl.ds(base, _BPW)], idx_sh.at[sid])
    pltpu.sync_copy(idx_sh.at[sid], idx_s)

    sems = [sem_a, sem_b]
    pending = [None, None]

    for g in range(_NG):
        gb = g * _K
        sem = sems[g % 2]
        if pending[g % 2] is not None:
            for c in pending[g % 2]:
                c.wait()
        copies = []
        for k in range(_K):
            v = idx_s[gb + k]
            t = lax.shift_right_logical(v, 3)
            r = lax.bitwise_and(v, _R - 1)
            copies.append(
                pltpu.async_copy(
                    table_hbm.at[t, r], rows_v.at[gb + k], sem
                )
            )
        pending[g % 2] = copies
    for p in pending:
        if p is not None:
            for c in p:
                c.wait()
    pltpu.sync_copy(rows_v, out_hbm.at[pl.ds(base, _BPW)])


def kernel(inputs, in_embed_weight):
    idx = inputs.astype(jnp.int32)
    table3 = in_embed_weight.reshape(_NT, _R, _D)
    return _gather_kernel(table3, idx)


# speedup vs baseline: 7.7951x; 1.0048x over previous
import functools

import jax
import jax.numpy as jnp
from jax import lax
from jax.experimental import pallas as pl
from jax.experimental.pallas import tpu as pltpu
from jax.experimental.pallas import tpu_sc as plsc

_D = 64
_B = 16384
_R = 8
_NT = 125000

_info = plsc.get_sparse_core_info()
_NC = _info.num_cores
_NS = _info.num_subcores
_NW = _NC * _NS
_BPW = _B // _NW
_K = 64
_NG = _BPW // _K

_mesh = plsc.VectorSubcoreMesh(core_axis_name="c", subcore_axis_name="s")


@functools.partial(
    pl.kernel,
    mesh=_mesh,
    out_type=jax.ShapeDtypeStruct((_B, _D), jnp.float32),
    scratch_types=[
        pltpu.SMEM((_BPW,), jnp.int32),
        pltpu.VMEM_SHARED((_NS, _BPW), jnp.int32),
        pltpu.VMEM((_BPW, _D), jnp.float32),
        pltpu.SemaphoreType.DMA,
        pltpu.SemaphoreType.DMA,
        pltpu.SemaphoreType.DMA,
    ],
)
def _gather_kernel(table_hbm, idx_hbm, out_hbm, idx_s, idx_sh, rows_v,
                   sem_a, sem_b, sem_o):
    sid = lax.axis_index("s")
    wid = sid * _NC + lax.axis_index("c")
    base = wid * _BPW
    pltpu.sync_copy(idx_hbm.at[pl.ds(base, _BPW)], idx_sh.at[sid])
    pltpu.sync_copy(idx_sh.at[sid], idx_s)

    sems = [sem_a, sem_b]
    pending = [None, None]
    out_copies = []

    def flush_group(g2):
        gb2 = g2 * _K
        out_copies.append(
            pltpu.async_copy(
                rows_v.at[pl.ds(gb2, _K)],
                out_hbm.at[pl.ds(base + gb2, _K)],
                sem_o,
            )
        )

    for g in range(_NG):
        gb = g * _K
        sem = sems[g % 2]
        if pending[g % 2] is not None:
            for c in pending[g % 2]:
                c.wait()
            flush_group(g - 2)
        copies = []
        for k in range(_K):
            v = idx_s[gb + k]
            t = lax.shift_right_logical(v, 3)
            r = lax.bitwise_and(v, _R - 1)
            copies.append(
                pltpu.async_copy(
                    table_hbm.at[t, r], rows_v.at[gb + k], sem
                )
            )
        pending[g % 2] = copies
    for i, p in enumerate(pending):
        if p is not None:
            for c in p:
                c.wait()
            flush_group(_NG - 2 + i)
    for c in out_copies:
        c.wait()


def kernel(inputs, in_embed_weight):
    idx = inputs.astype(jnp.int32)
    table3 = in_embed_weight.reshape(_NT, _R, _D)
    return _gather_kernel(table3, idx)
